# baseline (speedup 1.0000x reference)
import functools

import jax
import jax.numpy as jnp
from jax import lax
from jax.experimental import pallas as pl
from jax.experimental.pallas import tpu as pltpu
from jax.experimental.pallas import tpu_sc as plsc

D = 128
L = 16
NC = 2
NS = 16
NW = NC * NS
K = 128


def _sc_segment_sum(x_pad, row3, col3, n_pad, nch):
    rpt = n_pad // NS

    mesh = plsc.VectorSubcoreMesh(core_axis_name="c", subcore_axis_name="s")

    @functools.partial(
        pl.kernel,
        out_type=jax.ShapeDtypeStruct((NC, n_pad, D), jnp.float32),
        mesh=mesh,
        scratch_types=[
            pltpu.VMEM((nch, K), jnp.int32),
            pltpu.VMEM((K,), jnp.int32),
            pltpu.VMEM((K,), jnp.int32),
            pltpu.VMEM((K, D), jnp.float32),
            pltpu.VMEM((K, D), jnp.float32),
            pltpu.VMEM_SHARED((n_pad, D), jnp.float32),
            pltpu.SemaphoreType.DMA,
            pltpu.SemaphoreType.DMA,
            pltpu.SemaphoreType.DMA,
            pltpu.SemaphoreType.DMA,
        ],
    )
    def seg_sum(x_hbm, row_hbm, col_hbm, out_hbm,
                ci, ri0, ri1, b0, b1, acc_sh, gs0, gs1, is0, is1):
        cid = lax.axis_index("c")
        sid = lax.axis_index("s")
        wid = cid * NS + sid

        pltpu.sync_copy(col_hbm.at[wid], ci)

        zeros16 = jnp.zeros((L,), jnp.float32)

        def zero_row(r):
            for c in range(0, D, L):
                b0[r, pl.ds(c, L)] = zeros16

        pl.loop(0, K)(zero_row)

        base = sid * rpt
        off = 0
        while off < rpt:
            n = min(K, rpt - off)
            pltpu.sync_copy(b0.at[pl.ds(0, n)], acc_sh.at[pl.ds(base + off, n)])
            off += n

        pltpu.sync_copy(row_hbm.at[wid, 0], ri0)
        pltpu.async_copy(x_hbm.at[ri0], b0, gs0)
        pltpu.async_copy(row_hbm.at[wid, 1], ri1, is1)

        plsc.subcore_barrier()

        def ring(g):
            pltpu.make_async_copy(row_hbm.at[wid, 0], ri1, is1).wait()
            pltpu.async_copy(x_hbm.at[ri1], b1, gs1)
            pltpu.make_async_copy(x_hbm.at[ri0], b0, gs0).wait()
            pltpu.sync_copy(b0, acc_sh.at[ci.at[g]], add=True)
            pltpu.async_copy(row_hbm.at[wid, g + 2], ri0, is0)
            pltpu.make_async_copy(row_hbm.at[wid, 0], ri0, is0).wait()
            pltpu.async_copy(x_hbm.at[ri0], b0, gs0)
            pltpu.make_async_copy(x_hbm.at[ri1], b1, gs1).wait()
            pltpu.sync_copy(b1, acc_sh.at[ci.at[g + 1]], add=True)
            pltpu.async_copy(row_hbm.at[wid, g + 3], ri1, is1)

        pl.loop(0, nch, step=2)(ring)

        pltpu.make_async_copy(x_hbm.at[ri0], b0, gs0).wait()
        pltpu.make_async_copy(row_hbm.at[wid, 0], ri1, is1).wait()

        plsc.subcore_barrier()

        pltpu.sync_copy(acc_sh.at[pl.ds(base, rpt)],
                        out_hbm.at[cid, pl.ds(base, rpt)])

    return seg_sum(x_pad, row3, col3)


def _tc_normalize(partials, n):
    br = 1000
    assert n % br == 0

    def body(p_ref, o_ref):
        p = p_ref[0] + p_ref[1]
        lns = 2.0 * p[:, 0] * p[:, 0] - jnp.sum(p * p, axis=1)
        bad = lns <= 1e-8
        basepoint = (lax.broadcasted_iota(jnp.int32, (br, D), 1) == 0)
        p = jnp.where(bad[:, None], basepoint.astype(jnp.float32), p)
        lns = jnp.where(bad, 1.0, lns)
        denom = jnp.maximum(jnp.sqrt(jnp.maximum(lns, 0.0)), 1e-12)
        out = p / denom[:, None]
        o_ref[...] = jnp.where((out[:, 0] <= 0.0)[:, None], -out, out)

    return pl.pallas_call(
        body,
        grid=(n // br,),
        in_specs=[pl.BlockSpec((2, br, D), lambda i: (0, i, 0))],
        out_specs=pl.BlockSpec((br, D), lambda i: (i, 0)),
        out_shape=jax.ShapeDtypeStruct((n, D), jnp.float32),
    )(partials)


def kernel(x, edge_index):
    n = x.shape[0]
    e = edge_index.shape[1]

    nch = -(-e // (NW * K * 2)) * 2
    e_pad = NW * K * nch
    n_pad = -(-(n + 1) // (NS * 8)) * (NS * 8)

    x_pad = jnp.pad(x, ((0, n_pad - n), (0, 0)))

    row = edge_index[0]
    col = edge_index[1]
    pad = e_pad - e
    row3 = jnp.concatenate([row, jnp.zeros((pad,), jnp.int32)]).reshape(NW, nch, K)
    col3 = jnp.concatenate([col, jnp.full((pad,), n, jnp.int32)]).reshape(NW, nch, K)
    row3 = jnp.pad(row3, ((0, 0), (0, 2), (0, 0)))

    partials = _sc_segment_sum(x_pad, row3, col3, n_pad, nch)
    return _tc_normalize(partials[:, :n, :], n)


# speedup vs baseline: 1.6281x; 1.6281x over previous
import functools

import jax
import jax.numpy as jnp
from jax import lax
from jax.experimental import pallas as pl
from jax.experimental.pallas import tpu as pltpu
from jax.experimental.pallas import tpu_sc as plsc

D = 128
L = 16
NC = 2
NS = 16
NW = NC * NS
K = 128


def _sc_segment_sum(x_pad, row3, col3, n_pad, nch):
    rpt = n_pad // NS

    mesh = plsc.VectorSubcoreMesh(core_axis_name="c", subcore_axis_name="s")

    @functools.partial(
        pl.kernel,
        out_type=jax.ShapeDtypeStruct((NC, n_pad, D), jnp.float32),
        mesh=mesh,
        scratch_types=[
            pltpu.VMEM((nch, K), jnp.int32),
            pltpu.VMEM((nch, K), jnp.int32),
            pltpu.VMEM((K, D), jnp.float32),
            pltpu.VMEM_SHARED((n_pad, D), jnp.float32),
            pltpu.SemaphoreType.DMA,
        ],
    )
    def seg_sum(x_hbm, row_hbm, col_hbm, out_hbm, ri, ci, buf, acc_sh, gsem):
        cid = lax.axis_index("c")
        sid = lax.axis_index("s")
        wid = cid * NS + sid

        pltpu.sync_copy(row_hbm.at[wid], ri)
        pltpu.sync_copy(col_hbm.at[wid], ci)

        zeros16 = jnp.zeros((L,), jnp.float32)

        def zero_row(r):
            for c in range(0, D, L):
                buf[r, pl.ds(c, L)] = zeros16

        pl.loop(0, K)(zero_row)

        base = sid * rpt
        off = 0
        while off < rpt:
            n = min(K, rpt - off)
            pltpu.sync_copy(buf.at[pl.ds(0, n)], acc_sh.at[pl.ds(base + off, n)])
            off += n

        plsc.subcore_barrier()

        def chunk(j):
            pltpu.async_copy(x_hbm.at[ri.at[j]], buf, gsem).wait()
            pltpu.sync_copy(buf, acc_sh.at[ci.at[j]], add=True)

        pl.loop(0, nch)(chunk)

        plsc.subcore_barrier()

        pltpu.sync_copy(acc_sh.at[pl.ds(base, rpt)],
                        out_hbm.at[cid, pl.ds(base, rpt)])

    return seg_sum(x_pad, row3, col3)


def _tc_normalize(partials, n):
    br = 1000
    assert n % br == 0

    def body(p_ref, o_ref):
        p = p_ref[0] + p_ref[1]
        lns = 2.0 * p[:, 0] * p[:, 0] - jnp.sum(p * p, axis=1)
        bad = lns <= 1e-8
        basepoint = (lax.broadcasted_iota(jnp.int32, (br, D), 1) == 0)
        p = jnp.where(bad[:, None], basepoint.astype(jnp.float32), p)
        lns = jnp.where(bad, 1.0, lns)
        denom = jnp.maximum(jnp.sqrt(jnp.maximum(lns, 0.0)), 1e-12)
        out = p / denom[:, None]
        o_ref[...] = jnp.where((out[:, 0] <= 0.0)[:, None], -out, out)

    return pl.pallas_call(
        body,
        grid=(n // br,),
        in_specs=[pl.BlockSpec((2, br, D), lambda i: (0, i, 0))],
        out_specs=pl.BlockSpec((br, D), lambda i: (i, 0)),
        out_shape=jax.ShapeDtypeStruct((n, D), jnp.float32),
    )(partials)


def kernel(x, edge_index):
    n = x.shape[0]
    e = edge_index.shape[1]

    nch = -(-e // (NW * K))
    e_pad = NW * K * nch
    n_pad = -(-(n + 1) // (NS * 8)) * (NS * 8)

    x_pad = jnp.pad(x, ((0, n_pad - n), (0, 0)))

    row = edge_index[0]
    col = edge_index[1]
    pad = e_pad - e
    row3 = jnp.concatenate([row, jnp.zeros((pad,), jnp.int32)]).reshape(NW, nch, K)
    col3 = jnp.concatenate([col, jnp.full((pad,), n, jnp.int32)]).reshape(NW, nch, K)

    partials = _sc_segment_sum(x_pad, row3, col3, n_pad, nch)
    return _tc_normalize(partials[:, :n, :], n)


# speedup vs baseline: 2.7328x; 1.6785x over previous
import functools

import jax
import jax.numpy as jnp
from jax import lax
from jax.experimental import pallas as pl
from jax.experimental.pallas import tpu as pltpu
from jax.experimental.pallas import tpu_sc as plsc

D = 128
L = 16
NC = 2
NS = 16
NW = NC * NS
K = 128


def _sc_segment_sum(x_pad, row3, col3, n_pad, nch):
    rpt = n_pad // NS

    mesh = plsc.VectorSubcoreMesh(core_axis_name="c", subcore_axis_name="s")

    @functools.partial(
        pl.kernel,
        out_type=jax.ShapeDtypeStruct((NC, n_pad, D), jnp.float32),
        mesh=mesh,
        scratch_types=[
            pltpu.VMEM((nch, K), jnp.int32),
            pltpu.VMEM((nch, K), jnp.int32),
            pltpu.VMEM((K, D), jnp.float32),
            pltpu.VMEM_SHARED((n_pad, D), jnp.float32),
            pltpu.SemaphoreType.DMA,
        ],
    )
    def seg_sum(x_hbm, row_hbm, col_hbm, out_hbm, ri, ci, buf, acc_sh, gsem):
        cid = lax.axis_index("c")
        sid = lax.axis_index("s")
        wid = cid * NS + sid

        pltpu.sync_copy(row_hbm.at[wid], ri)
        pltpu.sync_copy(col_hbm.at[wid], ci)

        zeros16 = jnp.zeros((L,), jnp.float32)

        def zero_row(r):
            for c in range(0, D, L):
                buf[r, pl.ds(c, L)] = zeros16

        pl.loop(0, K)(zero_row)

        base = sid * rpt
        off = 0
        while off < rpt:
            n = min(K, rpt - off)
            pltpu.sync_copy(buf.at[pl.ds(0, n)], acc_sh.at[pl.ds(base + off, n)])
            off += n

        plsc.subcore_barrier()

        def chunk(j):
            pltpu.async_copy(x_hbm.at[ri.at[j]], buf, gsem).wait()
            pltpu.sync_copy(buf, acc_sh.at[ci.at[j]], add=True)

        pl.loop(0, nch)(chunk)

        plsc.subcore_barrier()

        pltpu.sync_copy(acc_sh.at[pl.ds(base, rpt)],
                        out_hbm.at[cid, pl.ds(base, rpt)])

    return seg_sum(x_pad, row3, col3)


def _tc_normalize(partials, n):
    br = 1000
    assert n % br == 0

    def body(p_ref, o_ref):
        p = p_ref[0] + p_ref[1]
        lns = 2.0 * p[:, 0] * p[:, 0] - jnp.sum(p * p, axis=1)
        bad = lns <= 1e-8
        basepoint = (lax.broadcasted_iota(jnp.int32, (br, D), 1) == 0)
        p = jnp.where(bad[:, None], basepoint.astype(jnp.float32), p)
        lns = jnp.where(bad, 1.0, lns)
        denom = jnp.maximum(jnp.sqrt(jnp.maximum(lns, 0.0)), 1e-12)
        out = p / denom[:, None]
        o_ref[...] = jnp.where((out[:, 0] <= 0.0)[:, None], -out, out)

    return pl.pallas_call(
        body,
        grid=(n // br,),
        in_specs=[pl.BlockSpec((2, br, D), lambda i: (0, i, 0))],
        out_specs=pl.BlockSpec((br, D), lambda i: (i, 0)),
        out_shape=jax.ShapeDtypeStruct((n, D), jnp.float32),
    )(partials)


def kernel(x, edge_index):
    n = x.shape[0]
    e = edge_index.shape[1]

    nch = -(-e // (NW * K))
    e_pad = NW * K * nch
    n_pad = -(-(n + 1) // (NS * 8)) * (NS * 8)

    x_pad = jnp.pad(x, ((0, n_pad - n), (0, 0)))

    row = edge_index[0]
    col = edge_index[1]
    pad = e_pad - e
    ar = jnp.arange(pad, dtype=jnp.int32)
    spare = n_pad - n
    row3 = jnp.concatenate([row, ar % n]).reshape(NW, nch, K)
    col3 = jnp.concatenate([col, n + ar % spare]).reshape(NW, nch, K)

    partials = _sc_segment_sum(x_pad, row3, col3, n_pad, nch)
    return _tc_normalize(partials[:, :n, :], n)


# speedup vs baseline: 2.8691x; 1.0499x over previous
import functools

import jax
import jax.numpy as jnp
from jax import lax
from jax.experimental import pallas as pl
from jax.experimental.pallas import tpu as pltpu
from jax.experimental.pallas import tpu_sc as plsc

D = 128
L = 16
NC = 2
NS = 16
NW = NC * NS
K = 128


def _sc_segment_sum(x_pad, row3, col3, n_pad, nch):
    rpt = n_pad // NS

    mesh = plsc.VectorSubcoreMesh(core_axis_name="c", subcore_axis_name="s")

    @functools.partial(
        pl.kernel,
        out_type=jax.ShapeDtypeStruct((NC, n_pad, D), jnp.float32),
        mesh=mesh,
        scratch_types=[
            pltpu.VMEM((nch, K), jnp.int32),
            pltpu.VMEM((nch, K), jnp.int32),
            pltpu.VMEM((K, D), jnp.float32),
            pltpu.VMEM_SHARED((n_pad, D), jnp.float32),
            pltpu.SemaphoreType.DMA,
        ],
    )
    def seg_sum(x_hbm, row_hbm, col_hbm, out_hbm, ri, ci, buf, acc_sh, gsem):
        cid = lax.axis_index("c")
        sid = lax.axis_index("s")
        wid = cid * NS + sid

        pltpu.sync_copy(row_hbm.at[wid], ri)
        pltpu.sync_copy(col_hbm.at[wid], ci)

        zeros16 = jnp.zeros((L,), jnp.float32)

        def zero_row(r):
            for c in range(0, D, L):
                buf[r, pl.ds(c, L)] = zeros16

        pl.loop(0, K)(zero_row)

        base = sid * rpt
        off = 0
        while off < rpt:
            n = min(K, rpt - off)
            pltpu.sync_copy(buf.at[pl.ds(0, n)], acc_sh.at[pl.ds(base + off, n)])
            off += n

        plsc.subcore_barrier()

        def chunk(j):
            pltpu.async_copy(x_hbm.at[ri.at[j]], buf, gsem).wait()
            pltpu.sync_copy(buf, acc_sh.at[ci.at[j]], add=True)

        pl.loop(0, nch)(chunk)

        plsc.subcore_barrier()

        pltpu.sync_copy(acc_sh.at[pl.ds(base, rpt)],
                        out_hbm.at[cid, pl.ds(base, rpt)])

    return seg_sum(x_pad, row3, col3)


def _tc_normalize(partials, n):
    br = 1000
    assert n % br == 0

    def body(p_ref, o_ref):
        p = p_ref[0] + p_ref[1]
        lns = 2.0 * p[:, 0] * p[:, 0] - jnp.sum(p * p, axis=1)
        bad = lns <= 1e-8
        basepoint = (lax.broadcasted_iota(jnp.int32, (br, D), 1) == 0)
        p = jnp.where(bad[:, None], basepoint.astype(jnp.float32), p)
        lns = jnp.where(bad, 1.0, lns)
        denom = jnp.maximum(jnp.sqrt(jnp.maximum(lns, 0.0)), 1e-12)
        out = p / denom[:, None]
        o_ref[...] = jnp.where((out[:, 0] <= 0.0)[:, None], -out, out)

    return pl.pallas_call(
        body,
        grid=(n // br,),
        in_specs=[pl.BlockSpec((2, br, D), lambda i: (0, i, 0))],
        out_specs=pl.BlockSpec((br, D), lambda i: (i, 0)),
        out_shape=jax.ShapeDtypeStruct((n, D), jnp.float32),
    )(partials)


def kernel(x, edge_index):
    n = x.shape[0]
    e = edge_index.shape[1]

    nch = -(-e // (NW * K))
    e_pad = NW * K * nch
    n_pad = -(-(n + 1) // (NS * 8)) * (NS * 8)


    row = edge_index[0]
    col = edge_index[1]
    pad = e_pad - e
    ar = jnp.arange(pad, dtype=jnp.int32)
    spare = n_pad - n
    row3 = jnp.concatenate([row, ar % n]).reshape(NW, nch, K)
    col3 = jnp.concatenate([col, n + ar % spare]).reshape(NW, nch, K)

    partials = _sc_segment_sum(x, row3, col3, n_pad, nch)
    return _tc_normalize(partials, n)


# speedup vs baseline: 3.9895x; 1.3905x over previous
import functools

import jax
import jax.numpy as jnp
from jax import lax
from jax.experimental import pallas as pl
from jax.experimental.pallas import tpu as pltpu
from jax.experimental.pallas import tpu_sc as plsc

D = 128
L = 16
NC = 2
NS = 16
NW = NC * NS
K = 128


def _sc_segment_sum(x_pad, row3, col3, n_pad, nch):
    rpt = n_pad // NS

    mesh = plsc.VectorSubcoreMesh(core_axis_name="c", subcore_axis_name="s")

    @functools.partial(
        pl.kernel,
        out_type=jax.ShapeDtypeStruct((NC, n_pad, D), jnp.float32),
        mesh=mesh,
        scratch_types=[
            pltpu.VMEM((nch // 2 + 2, K), jnp.int32),
            pltpu.VMEM((nch // 2, K), jnp.int32),
            pltpu.VMEM((K, D), jnp.float32),
            pltpu.VMEM((K, D), jnp.float32),
            pltpu.VMEM_SHARED((n_pad, D), jnp.float32),
            pltpu.SemaphoreType.DMA,
            pltpu.SemaphoreType.DMA,
        ],
    )
    def seg_sum(x_hbm, row_hbm, col_hbm, out_hbm,
                ri, ci, b0, b1, acc_sh, gs0, gs1):
        cid = lax.axis_index("c")
        sid = lax.axis_index("s")
        wid = cid * NS + sid
        half = nch // 2

        zeros16 = jnp.zeros((L,), jnp.float32)

        def zero_row(r):
            for c in range(0, D, L):
                b0[r, pl.ds(c, L)] = zeros16

        pl.loop(0, K)(zero_row)

        base = sid * rpt
        off = 0
        while off < rpt:
            n = min(K, rpt - off)
            pltpu.sync_copy(b0.at[pl.ds(0, n)], acc_sh.at[pl.ds(base + off, n)])
            off += n

        def ring(g):
            pltpu.make_async_copy(x_hbm.at[ri.at[g]], b0, gs0).wait()
            pltpu.sync_copy(b0, acc_sh.at[ci.at[g]], add=True)
            pltpu.async_copy(x_hbm.at[ri.at[g + 2]], b0, gs0)
            pltpu.make_async_copy(x_hbm.at[ri.at[g + 1]], b1, gs1).wait()
            pltpu.sync_copy(b1, acc_sh.at[ci.at[g + 1]], add=True)
            pltpu.async_copy(x_hbm.at[ri.at[g + 3]], b1, gs1)

        for p in range(2):
            pltpu.sync_copy(row_hbm.at[wid, p], ri)
            pltpu.sync_copy(col_hbm.at[wid, p], ci)
            pltpu.async_copy(x_hbm.at[ri.at[0]], b0, gs0)
            pltpu.async_copy(x_hbm.at[ri.at[1]], b1, gs1)
            if p == 0:
                plsc.subcore_barrier()
            pl.loop(0, half, step=2)(ring)
            pltpu.make_async_copy(x_hbm.at[ri.at[0]], b0, gs0).wait()
            pltpu.make_async_copy(x_hbm.at[ri.at[1]], b1, gs1).wait()

        plsc.subcore_barrier()

        pltpu.sync_copy(acc_sh.at[pl.ds(base, rpt)],
                        out_hbm.at[cid, pl.ds(base, rpt)])

    return seg_sum(x_pad, row3, col3)


def _tc_normalize(partials, n):
    br = 1000
    assert n % br == 0

    def body(p_ref, o_ref):
        p = p_ref[0] + p_ref[1]
        lns = 2.0 * p[:, 0] * p[:, 0] - jnp.sum(p * p, axis=1)
        bad = lns <= 1e-8
        basepoint = (lax.broadcasted_iota(jnp.int32, (br, D), 1) == 0)
        p = jnp.where(bad[:, None], basepoint.astype(jnp.float32), p)
        lns = jnp.where(bad, 1.0, lns)
        denom = jnp.maximum(jnp.sqrt(jnp.maximum(lns, 0.0)), 1e-12)
        out = p / denom[:, None]
        o_ref[...] = jnp.where((out[:, 0] <= 0.0)[:, None], -out, out)

    return pl.pallas_call(
        body,
        grid=(n // br,),
        in_specs=[pl.BlockSpec((2, br, D), lambda i: (0, i, 0))],
        out_specs=pl.BlockSpec((br, D), lambda i: (i, 0)),
        out_shape=jax.ShapeDtypeStruct((n, D), jnp.float32),
    )(partials)


def kernel(x, edge_index):
    n = x.shape[0]
    e = edge_index.shape[1]

    nch = -(-e // (NW * K * 4)) * 4
    e_pad = NW * K * nch
    n_pad = -(-(n + 1) // (NS * 8)) * (NS * 8)


    row = edge_index[0]
    col = edge_index[1]
    pad = e_pad - e
    ar = jnp.arange(pad, dtype=jnp.int32)
    spare = n_pad - n
    row3 = jnp.concatenate([row, ar % n]).reshape(NW, 2, nch // 2, K)
    col3 = jnp.concatenate([col, n + ar % spare]).reshape(NW, 2, nch // 2, K)
    extra = (jnp.arange(2 * K, dtype=jnp.int32) % n).reshape(1, 1, 2, K)
    row3 = jnp.concatenate(
        [row3, jnp.broadcast_to(extra, (NW, 2, 2, K))], axis=2)

    partials = _sc_segment_sum(x, row3, col3, n_pad, nch)
    return _tc_normalize(partials, n)
